# baseline (speedup 1.0000x reference)
import functools

import numpy as np
import jax
import jax.numpy as jnp
from jax import lax
from jax.experimental import pallas as pl
from jax.experimental.pallas import tpu as pltpu
from jax.experimental.pallas import tpu_sc as plsc

_SEGMENTS = ((0, 128, 1), (128, 128, 3), (512, 128, 5), (1152, 128, 7))
_D = 2048
_Z = 16384
_NW = 32
_ROWS_PER_W = _Z // _NW
_B = 8
_NBLK = _ROWS_PER_W // _B
_NIN = 3
_UNROLL = 12


def _build_perm() -> np.ndarray:
    p = np.empty(_D, np.int32)
    for off, u, v in _SEGMENTS:
        for k in range(v):
            for i in range(u):
                p[off + k * u + i] = off + i * v + k
    return p


_PERM = _build_perm()


@jax.jit
def _sc_transpose(x, perm):
    mesh = plsc.VectorSubcoreMesh(core_axis_name="c", subcore_axis_name="s")

    @functools.partial(
        pl.kernel,
        mesh=mesh,
        out_type=jax.ShapeDtypeStruct((_Z, _D), jnp.float32),
        scratch_types=(
            [pltpu.VMEM((_D,), jnp.int32)]
            + [pltpu.VMEM((_B, _D), jnp.float32)] * (_NIN + 2)
            + [pltpu.VMEM_SHARED((16, 2, _B, _D), jnp.float32)]
            + [pltpu.SemaphoreType.DMA] * (_NIN + 5)
        ),
        compiler_params=pltpu.CompilerParams(needs_layout_passes=False),
    )
    def k(x_hbm, perm_hbm, out_hbm, idx_tab, *rest):
        in_v = rest[:_NIN]
        out_v = rest[_NIN:_NIN + 2]
        spmem = rest[_NIN + 2]
        sems = rest[_NIN + 3:]
        sem_in = sems[:_NIN]
        sem_out = sems[_NIN]
        sem_sp1 = sems[_NIN + 1:_NIN + 3]
        sem_sp2 = sems[_NIN + 3:_NIN + 5]
        cid = lax.axis_index("c")
        sid = lax.axis_index("s")
        wid = sid * 2 + cid
        row0 = wid * _ROWS_PER_W

        def in_copy(g, b):
            return pltpu.make_async_copy(
                x_hbm.at[pl.ds(row0 + g * _B, _B)], in_v[b], sem_in[b])

        def out_copy(g):
            return pltpu.make_async_copy(
                out_v[0], out_hbm.at[pl.ds(row0 + g * _B, _B)], sem_out)

        def sp1_copy(s):
            return pltpu.make_async_copy(
                out_v[1], spmem.at[sid, s], sem_sp1[s])

        def sp2_copy(g, s):
            return pltpu.make_async_copy(
                spmem.at[sid, s],
                out_hbm.at[pl.ds(row0 + g * _B, _B)], sem_sp2[s])

        def compute(bin_, bout):
            @plsc.parallel_loop(0, _D, 16, unroll=4)
            def _(i):
                cols = idx_tab[pl.ds(i, 16)]
                for r in range(_B):
                    rv = jnp.full((16,), r, jnp.int32)
                    vals = plsc.load_gather(in_v[bin_], [rv, cols])
                    out_v[bout][r, pl.ds(i, 16)] = vals

        def step(g, t, static):
            def when(cond, fn):
                if static:
                    if cond:
                        fn()
                else:
                    pl.when(cond)(fn)

            b = t % _NIN
            in_copy(g, b).wait()
            if t % 2 == 1:
                s = (t // 2) % 2
                when(g >= 5, lambda: sp2_copy(g - 4, s).wait())
                when(g >= 3, lambda: sp1_copy(1 - s).wait())
                when(g >= 3, lambda: sp2_copy(g - 2, 1 - s).start())
                compute(b, 1)
                sp1_copy(s).start()
            else:
                when(g >= 2, lambda: out_copy(g - 2).wait())
                compute(b, 0)
                out_copy(g).start()
            when(g + _NIN < _NBLK, lambda: in_copy(g + _NIN, b).start())

        for b in range(_NIN):
            in_copy(b, b).start()
        pltpu.sync_copy(perm_hbm, idx_tab)

        def body(h, carry):
            g0 = h * _UNROLL
            for t in range(_UNROLL):
                step(g0 + t, t, static=False)
            return carry

        n_main = (_NBLK // _UNROLL) * _UNROLL
        lax.fori_loop(0, _NBLK // _UNROLL, body, 0)
        for g in range(n_main, _NBLK):
            step(g, g, static=True)

        out_copy(_NBLK - 2).wait()
        last = _NBLK - 1
        s_last = (last // 2) % 2
        sp1_copy(s_last).wait()
        sp2_copy(last, s_last).start()
        sp2_copy(last - 2, 1 - s_last).wait()
        sp2_copy(last, s_last).wait()

    return k(x, perm)


def kernel(x):
    return _sc_transpose(x, jnp.asarray(_PERM))


# speedup vs baseline: 1.0234x; 1.0234x over previous
import functools

import numpy as np
import jax
import jax.numpy as jnp
from jax import lax
from jax.experimental import pallas as pl
from jax.experimental.pallas import tpu as pltpu
from jax.experimental.pallas import tpu_sc as plsc

_SEGMENTS = ((0, 128, 1), (128, 128, 3), (512, 128, 5), (1152, 128, 7))
_D = 2048
_Z = 16384
_NW = 32
_ROWS_PER_W = _Z // _NW
_B = 8
_NBLK = _ROWS_PER_W // _B
_NBUF = 3


def _build_perm() -> np.ndarray:
    p = np.empty(_D, np.int32)
    for off, u, v in _SEGMENTS:
        for k in range(v):
            for i in range(u):
                p[off + k * u + i] = off + i * v + k
    return p


_PERM = _build_perm()


@jax.jit
def _sc_transpose(x, perm):
    mesh = plsc.VectorSubcoreMesh(core_axis_name="c", subcore_axis_name="s")

    @functools.partial(
        pl.kernel,
        mesh=mesh,
        out_type=jax.ShapeDtypeStruct((_Z, _D), jnp.float32),
        scratch_types=(
            [pltpu.VMEM((_D,), jnp.int32)]
            + [pltpu.VMEM((_B, _D), jnp.float32)] * (2 * _NBUF)
            + [pltpu.SemaphoreType.DMA] * (2 * _NBUF)
        ),
        compiler_params=pltpu.CompilerParams(needs_layout_passes=False),
    )
    def k(x_hbm, perm_hbm, out_hbm, idx_tab, *bufs):
        in_v = bufs[:_NBUF]
        out_v = bufs[_NBUF:2 * _NBUF]
        sem_in = bufs[2 * _NBUF:3 * _NBUF]
        sem_out = bufs[3 * _NBUF:4 * _NBUF]
        cid = lax.axis_index("c")
        sid = lax.axis_index("s")
        wid = sid * 2 + cid
        row0 = wid * _ROWS_PER_W

        def in_copy(g, b):
            return pltpu.make_async_copy(
                x_hbm.at[pl.ds(row0 + g * _B, _B)], in_v[b], sem_in[b])

        def out_copy(g, b):
            return pltpu.make_async_copy(
                out_v[b], out_hbm.at[pl.ds(row0 + g * _B, _B)], sem_out[b])

        def compute(b):
            @plsc.parallel_loop(0, _D, 16, unroll=4)
            def _(i):
                cols = idx_tab[pl.ds(i, 16)]
                for r in range(_B):
                    rv = jnp.full((16,), r, jnp.int32)
                    vals = plsc.load_gather(in_v[b], [rv, cols])
                    out_v[b][r, pl.ds(i, 16)] = vals

        for b in range(_NBUF):
            in_copy(b, b).start()
        pltpu.sync_copy(perm_hbm, idx_tab)

        def body(h, carry):
            for b in range(_NBUF):
                g = h * _NBUF + b
                in_copy(g, b).wait()

                @pl.when(g >= _NBUF)
                def _():
                    out_copy(g - _NBUF, b).wait()

                compute(b)
                out_copy(g, b).start()

                @pl.when(g + _NBUF < _NBLK)
                def _():
                    in_copy(g + _NBUF, b).start()
            return carry

        lax.fori_loop(0, _NBLK // _NBUF, body, 0)
        for g in range((_NBLK // _NBUF) * _NBUF, _NBLK):
            b = g % _NBUF
            in_copy(g, b).wait()
            if g >= _NBUF:
                out_copy(g - _NBUF, b).wait()
            compute(b)
            out_copy(g, b).start()
        for g in range(_NBLK - _NBUF, _NBLK):
            out_copy(g, g % _NBUF).wait()

    return k(x, perm)


def kernel(x):
    return _sc_transpose(x, jnp.asarray(_PERM))
